# Initial kernel scaffold; baseline (speedup 1.0000x reference)
import jax
import jax.numpy as jnp
from jax.experimental import pallas as pl


def kernel(atomic_numbers, positions, neighbors, neighbor_mask, neighbors_j, neighbors_k, triple_mask, emb, fd_W1, fd_b1, fd_W2, fd_b2, ft_W1, ft_b1, ft_W2, ft_b2, in2f_W, f2out_W, f2out_b, dense_W, dense_b):
    raise NotImplementedError("write your pallas kernel here")


# speedup vs baseline: 9.3096x; 9.3096x over previous
import numpy as np
import jax
import jax.numpy as jnp
from jax.experimental import pallas as pl

B, AT, NBR, NBRT = 4, 128, 32, 96
F = 128
NGD = 25
NGT = 25
NTH = 10
ZETA = 8.0
CUTOFF = 6.0
NINT = 3
MAXZ = 100

BA = B * AT
BLK = 16
NBLK = BA // BLK
ABLK = AT // BLK
RT = BLK * NBRT
RD = BLK * NBR
NGP = 32
NTP = 16
FTK = 256

_LOG2 = float(np.log(2.0))

_offs64 = np.linspace(0.001, CUTOFF - 0.5, NGT)
_WIDTH = float(_offs64[1] - _offs64[0])
_OFF_ROW = np.zeros((1, NGP), np.float32)
_OFF_ROW[0, :NGT] = _offs64
_theta = np.linspace(0.0, np.pi, NTH)
_CT_ROW = np.zeros((1, NTP), np.float32)
_ST_ROW = np.zeros((1, NTP), np.float32)
_CT_ROW[0, :NTH] = np.cos(_theta)
_ST_ROW[0, :NTH] = np.sin(_theta)
_EG = np.zeros((NGP, FTK), np.float32)
_ET = np.zeros((NTP, FTK), np.float32)
for _g in range(NGT):
    for _t in range(NTH):
        _EG[_g, _g * NTH + _t] = 1.0
        _ET[_t, _g * NTH + _t] = 1.0
_R96 = np.kron(np.eye(BLK, dtype=np.float32), np.ones((NBRT, 1), np.float32))
_R32 = np.kron(np.eye(BLK, dtype=np.float32), np.ones((NBR, 1), np.float32))
_SEGT = np.kron(np.eye(BLK, dtype=np.float32), np.ones((1, NBRT), np.float32))
_SEGD = np.kron(np.eye(BLK, dtype=np.float32), np.ones((1, NBR), np.float32))


def _ssp(x):
    return jnp.maximum(x, 0.0) + jnp.log1p(jnp.exp(-jnp.abs(x))) - _LOG2


def _onehot(idx_col, n):
    io = jax.lax.broadcasted_iota(jnp.int32, (idx_col.shape[0], n), 1)
    return (io == idx_col).astype(jnp.float32)


def _cutoff(r):
    return 0.5 * (jnp.cos(r * (np.pi / CUTOFF)) + 1.0) * (r < CUTOFF).astype(r.dtype)


def _filters_body(pos_ref, nbrd_ref, nbrj_ref, nbrk_ref, nmask_ref, tmask_ref,
                  off_ref, ct_ref, st_ref, eg_ref, et_ref, r96_ref, r32_ref,
                  fdw1_ref, fdb1_ref, fdw2_ref, fdb2_ref,
                  ftw1_ref, ftb1_ref, ftw2_ref, ftb2_ref,
                  wd_ref, wt_ref):
    g = pl.program_id(0)
    lb = (g % ABLK) * BLK
    pos = pos_ref[0]
    pos_blk = pos_ref[0, pl.ds(lb, BLK), :]
    off = off_ref[...]

    def gauss_pair(idx_ref, rmat, rows):
        oh = _onehot(idx_ref[0], AT)
        pj = jnp.dot(oh, pos, preferred_element_type=jnp.float32)
        pi = jnp.dot(rmat, pos_blk, preferred_element_type=jnp.float32)
        vec = pj - pi
        r = jnp.sqrt(jnp.sum(vec * vec, axis=-1, keepdims=True) + 1e-9)
        return r, vec

    r96 = r96_ref[...]
    r32 = r32_ref[...]

    rd, _ = gauss_pair(nbrd_ref, r32, RD)
    sd = (rd - off) * (1.0 / _WIDTH)
    fd = jnp.exp(-0.5 * sd * sd)
    cd = _cutoff(rd) * nmask_ref[0]

    rij, vij = gauss_pair(nbrj_ref, r96, RT)
    rik, vik = gauss_pair(nbrk_ref, r96, RT)
    cost = jnp.sum(vij * vik, axis=-1, keepdims=True) / (rij * rik)
    cost = jnp.clip(cost, -1.0 + 1e-6, 1.0 - 1e-6)
    sint = jnp.sqrt(1.0 - cost * cost)
    sij = (rij - off) * (1.0 / _WIDTH)
    sik = (rik - off) * (1.0 / _WIDTH)
    gr = jnp.exp(-0.5 * (sij * sij + sik * sik))
    base = 1.0 + cost * ct_ref[...] + sint * st_ref[...]
    b2 = base * base
    b4 = b2 * b2
    ang = (b4 * b4) * (2.0 ** (1.0 - ZETA))
    feat = (jnp.dot(gr, eg_ref[...], preferred_element_type=jnp.float32)
            * jnp.dot(ang, et_ref[...], preferred_element_type=jnp.float32))
    ct = _cutoff(rij) * _cutoff(rik) * tmask_ref[0]

    for i in range(NINT):
        hd = _ssp(jnp.dot(fd, fdw1_ref[i], preferred_element_type=jnp.float32)
                  + fdb1_ref[i])
        wd = (jnp.dot(hd, fdw2_ref[i], preferred_element_type=jnp.float32)
              + fdb2_ref[i]) * cd
        wd_ref[i] = wd
        ht = _ssp(jnp.dot(feat, ftw1_ref[i], preferred_element_type=jnp.float32)
                  + ftb1_ref[i])
        wt = (jnp.dot(ht, ftw2_ref[i], preferred_element_type=jnp.float32)
              + ftb2_ref[i]) * ct
        wt_ref[i] = wt


def _interact_body(x_ref, nbrd_ref, nbrj_ref, nbrk_ref, wd_ref, wt_ref,
                   in2f_ref, f2ow_ref, f2ob_ref, dw_ref, db_ref,
                   segt_ref, segd_ref, xo_ref):
    g = pl.program_id(0)
    lb = (g % ABLK) * BLK
    x = x_ref[0]
    y = jnp.dot(x, in2f_ref[...], preferred_element_type=jnp.float32)

    ohj = _onehot(nbrj_ref[0], AT)
    ohk = _onehot(nbrk_ref[0], AT)
    ohd = _onehot(nbrd_ref[0], AT)
    yj = jnp.dot(ohj, y, preferred_element_type=jnp.float32)
    yk = jnp.dot(ohk, y, preferred_element_type=jnp.float32)
    yd = jnp.dot(ohd, y, preferred_element_type=jnp.float32)
    prod_t = yj * yk * wt_ref[...]
    prod_d = yd * wd_ref[...]
    agg = (jnp.dot(segt_ref[...], prod_t, preferred_element_type=jnp.float32)
           + jnp.dot(segd_ref[...], prod_d, preferred_element_type=jnp.float32))
    v = _ssp(jnp.dot(agg, f2ow_ref[...], preferred_element_type=jnp.float32)
             + f2ob_ref[...])
    v = jnp.dot(v, dw_ref[...], preferred_element_type=jnp.float32) + db_ref[...]
    xo_ref[0] = x_ref[0, pl.ds(lb, BLK), :] + v


def kernel(atomic_numbers, positions, neighbors, neighbor_mask, neighbors_j,
           neighbors_k, triple_mask, emb, fd_W1, fd_b1, fd_W2, fd_b2,
           ft_W1, ft_b1, ft_W2, ft_b2, in2f_W, f2out_W, f2out_b,
           dense_W, dense_b):
    f32 = jnp.float32
    pos_pad = jnp.zeros((B, AT, 8), f32).at[:, :, :3].set(positions)
    nbrd = neighbors.astype(jnp.int32).reshape(NBLK, RD, 1)
    nbrj = neighbors_j.astype(jnp.int32).reshape(NBLK, RT, 1)
    nbrk = neighbors_k.astype(jnp.int32).reshape(NBLK, RT, 1)
    nmask = neighbor_mask.astype(f32).reshape(NBLK, RD, 1)
    tmask = triple_mask.astype(f32).reshape(NBLK, RT, 1)

    fd_W1p = jnp.zeros((NINT, NGP, F), f32).at[:, :NGD, :].set(fd_W1)
    ft_W1p = jnp.zeros((NINT, FTK, F), f32).at[:, :NGT * NTH, :].set(ft_W1)
    fd_b1r = fd_b1.reshape(NINT, 1, F)
    fd_b2r = fd_b2.reshape(NINT, 1, F)
    ft_b1r = ft_b1.reshape(NINT, 1, F)
    ft_b2r = ft_b2.reshape(NINT, 1, F)

    consts = dict(
        off=jnp.asarray(_OFF_ROW), ct=jnp.asarray(_CT_ROW),
        st=jnp.asarray(_ST_ROW), eg=jnp.asarray(_EG), et=jnp.asarray(_ET),
        r96=jnp.asarray(_R96), r32=jnp.asarray(_R32),
    )

    whole = lambda *shape: pl.BlockSpec(shape, lambda g: tuple(0 for _ in shape))

    wd_all, wt_all = pl.pallas_call(
        _filters_body,
        grid=(NBLK,),
        in_specs=[
            pl.BlockSpec((1, AT, 8), lambda g: (g // ABLK, 0, 0)),
            pl.BlockSpec((1, RD, 1), lambda g: (g, 0, 0)),
            pl.BlockSpec((1, RT, 1), lambda g: (g, 0, 0)),
            pl.BlockSpec((1, RT, 1), lambda g: (g, 0, 0)),
            pl.BlockSpec((1, RD, 1), lambda g: (g, 0, 0)),
            pl.BlockSpec((1, RT, 1), lambda g: (g, 0, 0)),
            whole(1, NGP), whole(1, NTP), whole(1, NTP),
            whole(NGP, FTK), whole(NTP, FTK),
            whole(RT, BLK), whole(RD, BLK),
            whole(NINT, NGP, F), whole(NINT, 1, F),
            whole(NINT, F, F), whole(NINT, 1, F),
            whole(NINT, FTK, F), whole(NINT, 1, F),
            whole(NINT, F, F), whole(NINT, 1, F),
        ],
        out_specs=[
            pl.BlockSpec((NINT, RD, F), lambda g: (0, g, 0)),
            pl.BlockSpec((NINT, RT, F), lambda g: (0, g, 0)),
        ],
        out_shape=[
            jax.ShapeDtypeStruct((NINT, BA * NBR, F), f32),
            jax.ShapeDtypeStruct((NINT, BA * NBRT, F), f32),
        ],
    )(pos_pad, nbrd, nbrj, nbrk, nmask, tmask,
      consts["off"], consts["ct"], consts["st"], consts["eg"], consts["et"],
      consts["r96"], consts["r32"],
      fd_W1p, fd_b1r, fd_W2, fd_b2r, ft_W1p, ft_b1r, ft_W2, ft_b2r)

    segt = jnp.asarray(_SEGT)
    segd = jnp.asarray(_SEGD)
    x = emb[atomic_numbers]

    for i in range(NINT):
        x = pl.pallas_call(
            _interact_body,
            grid=(NBLK,),
            in_specs=[
                pl.BlockSpec((1, AT, F), lambda g: (g // ABLK, 0, 0)),
                pl.BlockSpec((1, RD, 1), lambda g: (g, 0, 0)),
                pl.BlockSpec((1, RT, 1), lambda g: (g, 0, 0)),
                pl.BlockSpec((1, RT, 1), lambda g: (g, 0, 0)),
                pl.BlockSpec((RD, F), lambda g: (g, 0)),
                pl.BlockSpec((RT, F), lambda g: (g, 0)),
                whole(F, F), whole(F, F), whole(1, F),
                whole(F, F), whole(1, F),
                whole(BLK, RT), whole(BLK, RD),
            ],
            out_specs=pl.BlockSpec((1, BLK, F), lambda g: (g // ABLK, g % ABLK, 0)),
            out_shape=jax.ShapeDtypeStruct((B, AT, F), f32),
        )(x, nbrd, nbrj, nbrk, wd_all[i], wt_all[i],
          in2f_W[i], f2out_W[i], f2out_b[i].reshape(1, F),
          dense_W[i], dense_b[i].reshape(1, F), segt, segd)
    return x


# speedup vs baseline: 29.2935x; 3.1466x over previous
import functools

import numpy as np
import jax
import jax.numpy as jnp
from jax import lax
from jax.experimental import pallas as pl
from jax.experimental.pallas import tpu as pltpu
from jax.experimental.pallas import tpu_sc as plsc

B, AT, NBR, NBRT = 4, 128, 32, 96
F = 128
NGD = 25
NGT = 25
NTH = 10
ZETA = 8.0
CUTOFF = 6.0
NINT = 3
MAXZ = 100

BA = B * AT
RTB = AT * NBRT
RDB = AT * NBR
NCH = 2
RTC = RTB // NCH
ATC = AT // NCH
NGP = 32
NTP = 16
FTK = 256

_LOG2 = float(np.log(2.0))

_offs = np.linspace(0.001, CUTOFF - 0.5, NGT)
_W2 = float(_offs[1] - _offs[0]) ** 2
_OFFCOL = np.zeros((NGP, 1), np.float32)
_OFFCOL[:NGT, 0] = _offs
_theta = np.linspace(0.0, np.pi, NTH)
_CTCOL = np.zeros((NTP, 1), np.float32)
_STCOL = np.zeros((NTP, 1), np.float32)
_CTCOL[:NTH, 0] = np.cos(_theta)
_STCOL[:NTH, 0] = np.sin(_theta)
_EGT = np.zeros((FTK, NGP), np.float32)
_ETT = np.zeros((FTK, NTP), np.float32)
for _g in range(NGT):
    for _t in range(NTH):
        _EGT[_g * NTH + _t, _g] = 1.0
        _ETT[_g * NTH + _t, _t] = 1.0
_REPT = np.kron(np.eye(AT, dtype=np.float32), np.ones((1, NBRT), np.float32))
_REPD = np.kron(np.eye(AT, dtype=np.float32), np.ones((1, NBR), np.float32))
_SEGT = _REPT.T.copy()
_SEGD = _REPD.T.copy()


def _ssp(x):
    return jnp.maximum(x, 0.0) + jnp.log1p(jnp.exp(-jnp.abs(x))) - _LOG2


def _onehot_t(idx_row, n):
    io = jax.lax.broadcasted_iota(jnp.int32, (n, idx_row.shape[1]), 0)
    return (io == idx_row).astype(jnp.float32)


def _cutoff(r):
    return 0.5 * (jnp.cos(r * (np.pi / CUTOFF)) + 1.0) * (r < CUTOFF).astype(r.dtype)


def _dot(a, b):
    return jnp.dot(a, b, preferred_element_type=jnp.float32)


_SC_NC = 2
_SC_NS = 16
_NW = _SC_NC * _SC_NS
_EPW = BA // _NW


def _embed_sc_body(atn_hbm, emb_hbm, out_hbm, idx_v, rows_v, sem):
    wid = lax.axis_index("s") * _SC_NC + lax.axis_index("c")
    base = wid * _EPW
    pltpu.sync_copy(atn_hbm.at[pl.ds(base, _EPW)], idx_v)
    pltpu.async_copy(emb_hbm.at[idx_v], rows_v, sem).wait()
    pltpu.sync_copy(rows_v, out_hbm.at[pl.ds(base, _EPW)])


def _embed_sc(atn_flat, emb):
    k = functools.partial(
        pl.kernel,
        mesh=plsc.VectorSubcoreMesh(core_axis_name="c", subcore_axis_name="s"),
        out_type=jax.ShapeDtypeStruct((BA, F), jnp.float32),
        scratch_types=[
            pltpu.VMEM((_EPW,), jnp.int32),
            pltpu.VMEM((_EPW, F), jnp.float32),
            pltpu.SemaphoreType.DMA,
        ],
    )(_embed_sc_body)
    return k(atn_flat, emb)


def _fused_body(x0_ref, pos_ref, nbrd_ref, nbrj_ref, nbrk_ref,
                nmask_ref, tmask_ref,
                offc_ref, ctc_ref, stc_ref, egt_ref, ett_ref,
                rept_ref, repd_ref, segt_ref, segd_ref,
                fdw1_ref, fdb1_ref, fdw2_ref, fdb2_ref,
                ftw1_ref, ftb1_ref, ftw2_ref, ftb2_ref,
                in2f_ref, f2ow_ref, f2ob_ref, dw_ref, db_ref,
                xo_ref):
    pos_t = pos_ref[0]
    offc = offc_ref[...]

    ohd = _onehot_t(nbrd_ref[0], AT)
    pj_d = _dot(pos_t, ohd)
    pi_d = _dot(pos_t, repd_ref[...])
    vd = pj_d - pi_d
    rdst = jnp.sqrt(jnp.sum(vd * vd, axis=0, keepdims=True) + 1e-9)
    sdd = rdst - offc
    fd_t = jnp.exp((-0.5 / _W2) * sdd * sdd)
    cdf = _cutoff(rdst) * nmask_ref[0]

    ohj = _onehot_t(nbrj_ref[0], AT)
    ohk = _onehot_t(nbrk_ref[0], AT)
    pi_t = _dot(pos_t, rept_ref[...])
    vij = _dot(pos_t, ohj) - pi_t
    vik = _dot(pos_t, ohk) - pi_t
    rij = jnp.sqrt(jnp.sum(vij * vij, axis=0, keepdims=True) + 1e-9)
    rik = jnp.sqrt(jnp.sum(vik * vik, axis=0, keepdims=True) + 1e-9)
    cost = jnp.sum(vij * vik, axis=0, keepdims=True) / (rij * rik)
    cost = jnp.clip(cost, -1.0 + 1e-6, 1.0 - 1e-6)
    sint = jnp.sqrt(1.0 - cost * cost)
    sij = rij - offc
    sik = rik - offc
    gr_t = jnp.exp((-0.5 / _W2) * (sij * sij + sik * sik))
    base = 1.0 + ctc_ref[...] * cost + stc_ref[...] * sint
    b2 = base * base
    b4 = b2 * b2
    ang_t = (b4 * b4) * (2.0 ** (1.0 - ZETA))
    rboth = jnp.concatenate([rij, rik], axis=0)
    cutb = _cutoff(rboth)
    ctf = cutb[0:1, :] * cutb[1:2, :] * tmask_ref[0]

    x_t = x0_ref[0].T
    for i in range(NINT):
        y_t = _dot(in2f_ref[i], x_t)
        hd = _ssp(_dot(fdw1_ref[i], fd_t) + fdb1_ref[i])
        wd = (_dot(fdw2_ref[i], hd) + fdb2_ref[i]) * cdf
        prod_d = _dot(y_t, ohd) * wd
        agg = _dot(prod_d, segd_ref[...])
        for c in range(NCH):
            lo, hi = c * RTC, (c + 1) * RTC
            feat = (_dot(egt_ref[...], gr_t[:, lo:hi])
                    * _dot(ett_ref[...], ang_t[:, lo:hi]))
            ht = _ssp(_dot(ftw1_ref[i], feat) + ftb1_ref[i])
            wt = (_dot(ftw2_ref[i], ht) + ftb2_ref[i]) * ctf[:, lo:hi]
            prod_t = _dot(y_t, ohj[:, lo:hi]) * _dot(y_t, ohk[:, lo:hi]) * wt
            agg = agg + _dot(prod_t, segt_ref[lo:hi, :])
        v = _ssp(_dot(f2ow_ref[i], agg) + f2ob_ref[i])
        x_t = x_t + _dot(dw_ref[i], v) + db_ref[i]
    xo_ref[0] = x_t


def kernel(atomic_numbers, positions, neighbors, neighbor_mask, neighbors_j,
           neighbors_k, triple_mask, emb, fd_W1, fd_b1, fd_W2, fd_b2,
           ft_W1, ft_b1, ft_W2, ft_b2, in2f_W, f2out_W, f2out_b,
           dense_W, dense_b):
    f32 = jnp.float32
    pos_t = jnp.zeros((B, 8, AT), f32).at[:, :3, :].set(
        positions.transpose(0, 2, 1))
    nbrd_b = neighbors.astype(jnp.int32).reshape(B, 1, RDB)
    nbrj_b = neighbors_j.astype(jnp.int32).reshape(B, 1, RTB)
    nbrk_b = neighbors_k.astype(jnp.int32).reshape(B, 1, RTB)
    nmask_b = neighbor_mask.astype(f32).reshape(B, 1, RDB)
    tmask_b = triple_mask.astype(f32).reshape(B, 1, RTB)

    fd_W1p = jnp.zeros((NINT, NGP, F), f32).at[:, :NGD, :].set(fd_W1)
    ft_W1p = jnp.zeros((NINT, FTK, F), f32).at[:, :NGT * NTH, :].set(ft_W1)
    fdw1t = fd_W1p.transpose(0, 2, 1)
    ftw1t = ft_W1p.transpose(0, 2, 1)
    fdw2t = fd_W2.transpose(0, 2, 1)
    ftw2t = ft_W2.transpose(0, 2, 1)
    in2ft = in2f_W.transpose(0, 2, 1)
    f2owt = f2out_W.transpose(0, 2, 1)
    dwt = dense_W.transpose(0, 2, 1)
    fdb1c = fd_b1.reshape(NINT, F, 1)
    fdb2c = fd_b2.reshape(NINT, F, 1)
    ftb1c = ft_b1.reshape(NINT, F, 1)
    ftb2c = ft_b2.reshape(NINT, F, 1)
    f2obc = f2out_b.reshape(NINT, F, 1)
    dbc = dense_b.reshape(NINT, F, 1)

    x0 = _embed_sc(atomic_numbers.astype(jnp.int32).reshape(BA),
                   emb.astype(f32)).reshape(B, AT, F)

    whole = lambda *shape: pl.BlockSpec(shape, lambda g: tuple(0 for _ in shape))
    perb = lambda *shape: pl.BlockSpec((1,) + shape, lambda g: (g,) + tuple(
        0 for _ in shape))

    x_t = pl.pallas_call(
        _fused_body,
        grid=(B,),
        in_specs=[
            perb(AT, F), perb(8, AT),
            perb(1, RDB), perb(1, RTB), perb(1, RTB),
            perb(1, RDB), perb(1, RTB),
            whole(NGP, 1), whole(NTP, 1), whole(NTP, 1),
            whole(FTK, NGP), whole(FTK, NTP),
            whole(AT, RTB), whole(AT, RDB),
            whole(RTB, AT), whole(RDB, AT),
            whole(NINT, F, NGP), whole(NINT, F, 1),
            whole(NINT, F, F), whole(NINT, F, 1),
            whole(NINT, F, FTK), whole(NINT, F, 1),
            whole(NINT, F, F), whole(NINT, F, 1),
            whole(NINT, F, F), whole(NINT, F, F), whole(NINT, F, 1),
            whole(NINT, F, F), whole(NINT, F, 1),
        ],
        out_specs=pl.BlockSpec((1, F, AT), lambda g: (g, 0, 0)),
        out_shape=jax.ShapeDtypeStruct((B, F, AT), f32),
    )(x0, pos_t, nbrd_b, nbrj_b, nbrk_b, nmask_b, tmask_b,
      jnp.asarray(_OFFCOL), jnp.asarray(_CTCOL), jnp.asarray(_STCOL),
      jnp.asarray(_EGT), jnp.asarray(_ETT),
      jnp.asarray(_REPT), jnp.asarray(_REPD),
      jnp.asarray(_SEGT), jnp.asarray(_SEGD),
      fdw1t, fdb1c, fdw2t, fdb2c, ftw1t, ftb1c, ftw2t, ftb2c,
      in2ft, f2owt, f2obc, dwt, dbc)
    return x_t.transpose(0, 2, 1)


# speedup vs baseline: 29.4691x; 1.0060x over previous
import functools

import numpy as np
import jax
import jax.numpy as jnp
from jax import lax
from jax.experimental import pallas as pl
from jax.experimental.pallas import tpu as pltpu
from jax.experimental.pallas import tpu_sc as plsc

B, AT, NBR, NBRT = 4, 128, 32, 96
F = 128
NGD = 25
NGT = 25
NTH = 10
ZETA = 8.0
CUTOFF = 6.0
NINT = 3
MAXZ = 100

BA = B * AT
RTB = AT * NBRT
RDB = AT * NBR
NCH = 2
RTC = RTB // NCH
ATC = AT // NCH
NGP = 32
NTP = 16
FTK = 256

_LOG2 = float(np.log(2.0))

_offs = np.linspace(0.001, CUTOFF - 0.5, NGT)
_W2 = float(_offs[1] - _offs[0]) ** 2
_OFFCOL = np.zeros((NGP, 1), np.float32)
_OFFCOL[:NGT, 0] = _offs
_theta = np.linspace(0.0, np.pi, NTH)
_CTCOL = np.zeros((NTP, 1), np.float32)
_STCOL = np.zeros((NTP, 1), np.float32)
_CTCOL[:NTH, 0] = np.cos(_theta)
_STCOL[:NTH, 0] = np.sin(_theta)
_EGT = np.zeros((FTK, NGP), np.float32)
_ETT = np.zeros((FTK, NTP), np.float32)
for _g in range(NGT):
    for _t in range(NTH):
        _EGT[_g * NTH + _t, _g] = 1.0
        _ETT[_g * NTH + _t, _t] = 1.0
_REPT = np.kron(np.eye(AT, dtype=np.float32), np.ones((1, NBRT), np.float32))
_REPD = np.kron(np.eye(AT, dtype=np.float32), np.ones((1, NBR), np.float32))
_SEGT = _REPT.T.copy()
_SEGD = _REPD.T.copy()


def _ssp(x):
    return jnp.maximum(x, 0.0) + jnp.log1p(jnp.exp(-jnp.abs(x))) - _LOG2


def _onehot_t(idx_row, n, dtype=jnp.float32):
    io = jax.lax.broadcasted_iota(jnp.int32, (n, idx_row.shape[1]), 0)
    return (io == idx_row).astype(dtype)


def _cutoff(r):
    return 0.5 * (jnp.cos(r * (np.pi / CUTOFF)) + 1.0) * (r < CUTOFF).astype(r.dtype)


def _dot(a, b):
    return jnp.dot(a, b, preferred_element_type=jnp.float32)


_SC_NC = 2
_SC_NS = 16
_NW = _SC_NC * _SC_NS
_EPW = BA // _NW


def _embed_sc_body(atn_hbm, emb_hbm, out_hbm, idx_v, rows_v, sem):
    wid = lax.axis_index("s") * _SC_NC + lax.axis_index("c")
    base = wid * _EPW
    pltpu.sync_copy(atn_hbm.at[pl.ds(base, _EPW)], idx_v)
    pltpu.async_copy(emb_hbm.at[idx_v], rows_v, sem).wait()
    pltpu.sync_copy(rows_v, out_hbm.at[pl.ds(base, _EPW)])


def _embed_sc(atn_flat, emb):
    k = functools.partial(
        pl.kernel,
        mesh=plsc.VectorSubcoreMesh(core_axis_name="c", subcore_axis_name="s"),
        out_type=jax.ShapeDtypeStruct((BA, F), jnp.float32),
        scratch_types=[
            pltpu.VMEM((_EPW,), jnp.int32),
            pltpu.VMEM((_EPW, F), jnp.float32),
            pltpu.SemaphoreType.DMA,
        ],
    )(_embed_sc_body)
    return k(atn_flat, emb)


def _fused_body(x0_ref, pos_ref, nbrd_ref, nbrj_ref, nbrk_ref,
                nmask_ref, tmask_ref,
                offc_ref, ctc_ref, stc_ref, egt_ref, ett_ref,
                rept_ref, repd_ref, segt_ref, segd_ref,
                fdw1_ref, fdb1_ref, fdw2_ref, fdb2_ref,
                ftw1_ref, ftb1_ref, ftw2_ref, ftb2_ref,
                in2f_ref, f2ow_ref, f2ob_ref, dw_ref, db_ref,
                xo_ref):
    pos_t = pos_ref[0]
    offc = offc_ref[...]

    bf16 = jnp.bfloat16

    ohd = _onehot_t(nbrd_ref[0], AT)
    pj_d = _dot(pos_t, ohd)
    pi_d = _dot(pos_t, repd_ref[...])
    vd = pj_d - pi_d
    rdst = jnp.sqrt(jnp.sum(vd * vd, axis=0, keepdims=True) + 1e-9)
    sdd = rdst - offc
    fd_t = jnp.exp((-0.5 / _W2) * sdd * sdd)
    cdf = _cutoff(rdst) * nmask_ref[0]

    ohj = _onehot_t(nbrj_ref[0], AT)
    ohk = _onehot_t(nbrk_ref[0], AT)
    pi_t = _dot(pos_t, rept_ref[...])
    vij = _dot(pos_t, ohj) - pi_t
    vik = _dot(pos_t, ohk) - pi_t
    rij = jnp.sqrt(jnp.sum(vij * vij, axis=0, keepdims=True) + 1e-9)
    rik = jnp.sqrt(jnp.sum(vik * vik, axis=0, keepdims=True) + 1e-9)
    cost = jnp.sum(vij * vik, axis=0, keepdims=True) / (rij * rik)
    cost = jnp.clip(cost, -1.0 + 1e-6, 1.0 - 1e-6)
    sint = jnp.sqrt(1.0 - cost * cost)
    sij = rij - offc
    sik = rik - offc
    gr_t = jnp.exp((-0.5 / _W2) * (sij * sij + sik * sik))
    base = 1.0 + ctc_ref[...] * cost + stc_ref[...] * sint
    b2 = base * base
    b4 = b2 * b2
    ang_t = (b4 * b4) * (2.0 ** (1.0 - ZETA))
    rboth = jnp.concatenate([rij, rik], axis=0)
    cutb = _cutoff(rboth)
    ctf = cutb[0:1, :] * cutb[1:2, :] * tmask_ref[0]

    ohd_b = ohd.astype(bf16)
    ohj_b = ohj.astype(bf16)
    ohk_b = ohk.astype(bf16)
    gr_b = gr_t.astype(bf16)
    ang_b = ang_t.astype(bf16)
    fd_b = fd_t.astype(bf16)

    x_t = x0_ref[0].T
    for i in range(NINT):
        y_t = _dot(in2f_ref[i], x_t)
        y_b = y_t.astype(bf16)
        hd = _ssp(_dot(fdw1_ref[i], fd_b) + fdb1_ref[i])
        wd = (_dot(fdw2_ref[i], hd.astype(bf16)) + fdb2_ref[i]) * cdf
        prod_d = (_dot(y_b, ohd_b) * wd).astype(bf16)
        agg = _dot(prod_d, segd_ref[...])
        for c in range(NCH):
            lo, hi = c * RTC, (c + 1) * RTC
            feat = (_dot(egt_ref[...], gr_b[:, lo:hi])
                    * _dot(ett_ref[...], ang_b[:, lo:hi])).astype(bf16)
            ht = _ssp(_dot(ftw1_ref[i], feat) + ftb1_ref[i])
            wt = (_dot(ftw2_ref[i], ht.astype(bf16)) + ftb2_ref[i]) * ctf[:, lo:hi]
            prod_t = (_dot(y_b, ohj_b[:, lo:hi]) * _dot(y_b, ohk_b[:, lo:hi])
                      * wt).astype(bf16)
            agg = agg + _dot(prod_t, segt_ref[lo:hi, :])
        v = _ssp(_dot(f2ow_ref[i], agg) + f2ob_ref[i])
        x_t = x_t + _dot(dw_ref[i], v) + db_ref[i]
    xo_ref[0] = x_t


def kernel(atomic_numbers, positions, neighbors, neighbor_mask, neighbors_j,
           neighbors_k, triple_mask, emb, fd_W1, fd_b1, fd_W2, fd_b2,
           ft_W1, ft_b1, ft_W2, ft_b2, in2f_W, f2out_W, f2out_b,
           dense_W, dense_b):
    f32 = jnp.float32
    pos_t = jnp.zeros((B, 8, AT), f32).at[:, :3, :].set(
        positions.transpose(0, 2, 1))
    nbrd_b = neighbors.astype(jnp.int32).reshape(B, 1, RDB)
    nbrj_b = neighbors_j.astype(jnp.int32).reshape(B, 1, RTB)
    nbrk_b = neighbors_k.astype(jnp.int32).reshape(B, 1, RTB)
    nmask_b = neighbor_mask.astype(f32).reshape(B, 1, RDB)
    tmask_b = triple_mask.astype(f32).reshape(B, 1, RTB)

    fd_W1p = jnp.zeros((NINT, NGP, F), f32).at[:, :NGD, :].set(fd_W1)
    ft_W1p = jnp.zeros((NINT, FTK, F), f32).at[:, :NGT * NTH, :].set(ft_W1)
    bf16 = jnp.bfloat16
    fdw1t = fd_W1p.transpose(0, 2, 1).astype(bf16)
    ftw1t = ft_W1p.transpose(0, 2, 1).astype(bf16)
    fdw2t = fd_W2.transpose(0, 2, 1).astype(bf16)
    ftw2t = ft_W2.transpose(0, 2, 1).astype(bf16)
    in2ft = in2f_W.transpose(0, 2, 1)
    f2owt = f2out_W.transpose(0, 2, 1)
    dwt = dense_W.transpose(0, 2, 1)
    fdb1c = fd_b1.reshape(NINT, F, 1)
    fdb2c = fd_b2.reshape(NINT, F, 1)
    ftb1c = ft_b1.reshape(NINT, F, 1)
    ftb2c = ft_b2.reshape(NINT, F, 1)
    f2obc = f2out_b.reshape(NINT, F, 1)
    dbc = dense_b.reshape(NINT, F, 1)

    x0 = _embed_sc(atomic_numbers.astype(jnp.int32).reshape(BA),
                   emb.astype(f32)).reshape(B, AT, F)

    whole = lambda *shape: pl.BlockSpec(shape, lambda g: tuple(0 for _ in shape))
    perb = lambda *shape: pl.BlockSpec((1,) + shape, lambda g: (g,) + tuple(
        0 for _ in shape))

    x_t = pl.pallas_call(
        _fused_body,
        grid=(B,),
        in_specs=[
            perb(AT, F), perb(8, AT),
            perb(1, RDB), perb(1, RTB), perb(1, RTB),
            perb(1, RDB), perb(1, RTB),
            whole(NGP, 1), whole(NTP, 1), whole(NTP, 1),
            whole(FTK, NGP), whole(FTK, NTP),
            whole(AT, RTB), whole(AT, RDB),
            whole(RTB, AT), whole(RDB, AT),
            whole(NINT, F, NGP), whole(NINT, F, 1),
            whole(NINT, F, F), whole(NINT, F, 1),
            whole(NINT, F, FTK), whole(NINT, F, 1),
            whole(NINT, F, F), whole(NINT, F, 1),
            whole(NINT, F, F), whole(NINT, F, F), whole(NINT, F, 1),
            whole(NINT, F, F), whole(NINT, F, 1),
        ],
        out_specs=pl.BlockSpec((1, F, AT), lambda g: (g, 0, 0)),
        out_shape=jax.ShapeDtypeStruct((B, F, AT), f32),
    )(x0, pos_t, nbrd_b, nbrj_b, nbrk_b, nmask_b, tmask_b,
      jnp.asarray(_OFFCOL), jnp.asarray(_CTCOL), jnp.asarray(_STCOL),
      jnp.asarray(_EGT).astype(bf16), jnp.asarray(_ETT).astype(bf16),
      jnp.asarray(_REPT), jnp.asarray(_REPD),
      jnp.asarray(_SEGT).astype(bf16), jnp.asarray(_SEGD).astype(bf16),
      fdw1t, fdb1c, fdw2t, fdb2c, ftw1t, ftb1c, ftw2t, ftb2c,
      in2ft, f2owt, f2obc, dwt, dbc)
    return x_t.transpose(0, 2, 1)


# speedup vs baseline: 31.5917x; 1.0720x over previous
import functools

import numpy as np
import jax
import jax.numpy as jnp
from jax import lax
from jax.experimental import pallas as pl
from jax.experimental.pallas import tpu as pltpu
from jax.experimental.pallas import tpu_sc as plsc

B, AT, NBR, NBRT = 4, 128, 32, 96
F = 128
NGD = 25
NGT = 25
NTH = 10
ZETA = 8.0
CUTOFF = 6.0
NINT = 3
MAXZ = 100

BA = B * AT
RTB = AT * NBRT
RDB = AT * NBR
NCH = 2
RTC = RTB // NCH
FTW = NGT * NTH

_LOG2 = float(np.log(2.0))

_offs = np.linspace(0.001, CUTOFF - 0.5, NGT)
_W2 = float(_offs[1] - _offs[0]) ** 2
_OFFCOL = _offs.reshape(NGT, 1).astype(np.float32)
_theta = np.linspace(0.0, np.pi, NTH)
_CTCOL = np.cos(_theta).reshape(NTH, 1).astype(np.float32)
_STCOL = np.sin(_theta).reshape(NTH, 1).astype(np.float32)
_EGT = np.zeros((FTW, NGT), np.float32)
_ETT = np.zeros((FTW, NTH), np.float32)
for _g in range(NGT):
    for _t in range(NTH):
        _EGT[_g * NTH + _t, _g] = 1.0
        _ETT[_g * NTH + _t, _t] = 1.0
_REPT = np.kron(np.eye(AT, dtype=np.float32), np.ones((1, NBRT), np.float32))
_REPD = np.kron(np.eye(AT, dtype=np.float32), np.ones((1, NBR), np.float32))
_SEGT = _REPT.T.copy()
_SEGD = _REPD.T.copy()


def _ssp(x):
    return jnp.maximum(x, 0.0) + jnp.log1p(jnp.exp(-jnp.abs(x))) - _LOG2


def _onehot_t(idx_row, n, dtype=jnp.float32):
    io = jax.lax.broadcasted_iota(jnp.int32, (n, idx_row.shape[1]), 0)
    return (io == idx_row).astype(dtype)


def _cutoff(r):
    return 0.5 * (jnp.cos(r * (np.pi / CUTOFF)) + 1.0) * (r < CUTOFF).astype(r.dtype)


def _dot(a, b):
    return jnp.dot(a, b, preferred_element_type=jnp.float32)


_SC_NC = 2
_SC_NS = 16
_NW = _SC_NC * _SC_NS
_EPW = BA // _NW


def _embed_sc_body(atn_hbm, emb_hbm, out_hbm, idx_v, rows_v, sem):
    wid = lax.axis_index("s") * _SC_NC + lax.axis_index("c")
    base = wid * _EPW
    pltpu.sync_copy(atn_hbm.at[pl.ds(base, _EPW)], idx_v)
    pltpu.async_copy(emb_hbm.at[idx_v], rows_v, sem).wait()
    pltpu.sync_copy(rows_v, out_hbm.at[pl.ds(base, _EPW)])


def _embed_sc(atn_flat, emb):
    k = functools.partial(
        pl.kernel,
        mesh=plsc.VectorSubcoreMesh(core_axis_name="c", subcore_axis_name="s"),
        out_type=jax.ShapeDtypeStruct((BA, F), jnp.float32),
        scratch_types=[
            pltpu.VMEM((_EPW,), jnp.int32),
            pltpu.VMEM((_EPW, F), jnp.float32),
            pltpu.SemaphoreType.DMA,
        ],
    )(_embed_sc_body)
    return k(atn_flat, emb)


def _fused_body(x0_ref, pos_ref, nbrd_ref, nbrj_ref, nbrk_ref,
                nmask_ref, tmask_ref,
                offc_ref, ctc_ref, stc_ref, egt_ref, ett_ref,
                rept_ref, repd_ref, segt_ref, segd_ref,
                fdw1_ref, fdb1_ref, fdw2_ref, fdb2_ref,
                ftw1_ref, ftb1_ref, ftw2_ref, ftb2_ref,
                in2f_ref, f2ow_ref, f2ob_ref, dw_ref, db_ref,
                xo_ref):
    bf16 = jnp.bfloat16
    f32 = jnp.float32
    pos3 = pos_ref[0]
    pos_t = jnp.concatenate(
        [pos3, jnp.zeros((AT, 5), f32)], axis=1).T
    offc = offc_ref[...]

    bias_cols = jnp.concatenate(
        [fdb1_ref[...], fdb2_ref[...], ftb1_ref[...], ftb2_ref[...],
         f2ob_ref[...], db_ref[...]], axis=0).T

    def bcol(j):
        return bias_cols[:, j:j + 1]

    ohd = _onehot_t(nbrd_ref[0], AT)
    pj_d = _dot(pos_t, ohd)
    pi_d = _dot(pos_t, repd_ref[...])
    vd = pj_d - pi_d
    rdst = jnp.sqrt(jnp.sum(vd * vd, axis=0, keepdims=True) + 1e-9)
    sdd = rdst - offc
    fd_b = jnp.exp((-0.5 / _W2) * sdd * sdd).astype(bf16)
    cdf = _cutoff(rdst) * nmask_ref[0]

    ohj = _onehot_t(nbrj_ref[0], AT)
    ohk = _onehot_t(nbrk_ref[0], AT)
    pi_t = _dot(pos_t, rept_ref[...])
    vij = _dot(pos_t, ohj) - pi_t
    vik = _dot(pos_t, ohk) - pi_t
    rij = jnp.sqrt(jnp.sum(vij * vij, axis=0, keepdims=True) + 1e-9)
    rik = jnp.sqrt(jnp.sum(vik * vik, axis=0, keepdims=True) + 1e-9)
    cost = jnp.sum(vij * vik, axis=0, keepdims=True) / (rij * rik)
    cost = jnp.clip(cost, -1.0 + 1e-6, 1.0 - 1e-6)
    sint = jnp.sqrt(1.0 - cost * cost)
    sij = rij - offc
    sik = rik - offc
    gr_b = jnp.exp((-0.5 / _W2) * (sij * sij + sik * sik)).astype(bf16)
    base = 1.0 + ctc_ref[...] * cost + stc_ref[...] * sint
    b2 = base * base
    b4 = b2 * b2
    ang_b = ((b4 * b4) * (2.0 ** (1.0 - ZETA))).astype(bf16)
    rboth = jnp.concatenate([rij, rik], axis=0)
    cutb = _cutoff(rboth)
    ctf = cutb[0:1, :] * cutb[1:2, :] * tmask_ref[0]

    ohd_b = ohd.astype(bf16)
    ohj_b = ohj.astype(bf16)
    ohk_b = ohk.astype(bf16)
    egt_b = egt_ref[...].astype(bf16)
    ett_b = ett_ref[...].astype(bf16)
    segt_b = segt_ref[...]
    segd_b = segd_ref[...]

    x_t = x0_ref[...].T
    for i in range(NINT):
        fdw1t = fdw1_ref[i].T.astype(bf16)
        fdw2t = fdw2_ref[i].T.astype(bf16)
        ftw1t = ftw1_ref[i].T.astype(bf16)
        ftw2t = ftw2_ref[i].T.astype(bf16)
        in2ft = in2f_ref[i].T
        f2owt = f2ow_ref[i].T
        dwt = dw_ref[i].T
        y_t = _dot(in2ft, x_t)
        y_b = y_t.astype(bf16)
        hd = _ssp(_dot(fdw1t, fd_b) + bcol(i))
        wd = (_dot(fdw2t, hd.astype(bf16)) + bcol(NINT + i)) * cdf
        prod_d = (_dot(y_b, ohd_b) * wd).astype(bf16)
        agg = _dot(prod_d, segd_b)
        for c in range(NCH):
            lo, hi = c * RTC, (c + 1) * RTC
            feat = (_dot(egt_b, gr_b[:, lo:hi])
                    * _dot(ett_b, ang_b[:, lo:hi])).astype(bf16)
            ht = _ssp(_dot(ftw1t, feat) + bcol(2 * NINT + i))
            wt = (_dot(ftw2t, ht.astype(bf16))
                  + bcol(3 * NINT + i)) * ctf[:, lo:hi]
            prod_t = (_dot(y_b, ohj_b[:, lo:hi]) * _dot(y_b, ohk_b[:, lo:hi])
                      * wt).astype(bf16)
            agg = agg + _dot(prod_t, segt_b[lo:hi, :])
        v = _ssp(_dot(f2owt, agg) + bcol(4 * NINT + i))
        x_t = x_t + _dot(dwt, v) + bcol(5 * NINT + i)
    xo_ref[0] = x_t.T


def kernel(atomic_numbers, positions, neighbors, neighbor_mask, neighbors_j,
           neighbors_k, triple_mask, emb, fd_W1, fd_b1, fd_W2, fd_b2,
           ft_W1, ft_b1, ft_W2, ft_b2, in2f_W, f2out_W, f2out_b,
           dense_W, dense_b):
    f32 = jnp.float32
    bf16 = jnp.bfloat16
    nbrd_b = neighbors.astype(jnp.int32).reshape(B, 1, RDB)
    nbrj_b = neighbors_j.astype(jnp.int32).reshape(B, 1, RTB)
    nbrk_b = neighbors_k.astype(jnp.int32).reshape(B, 1, RTB)
    nmask_b = neighbor_mask.astype(f32).reshape(B, 1, RDB)
    tmask_b = triple_mask.astype(f32).reshape(B, 1, RTB)

    x0 = _embed_sc(atomic_numbers.astype(jnp.int32).reshape(BA),
                   emb.astype(f32))

    whole = lambda *shape: pl.BlockSpec(shape, lambda g: tuple(0 for _ in shape))
    perb = lambda *shape: pl.BlockSpec((1,) + shape, lambda g: (g,) + tuple(
        0 for _ in shape))

    out = pl.pallas_call(
        _fused_body,
        grid=(B,),
        in_specs=[
            pl.BlockSpec((AT, F), lambda g: (g, 0)),
            perb(AT, 3),
            perb(1, RDB), perb(1, RTB), perb(1, RTB),
            perb(1, RDB), perb(1, RTB),
            whole(NGT, 1), whole(NTH, 1), whole(NTH, 1),
            whole(FTW, NGT), whole(FTW, NTH),
            whole(AT, RTB), whole(AT, RDB),
            whole(RTB, AT), whole(RDB, AT),
            whole(NINT, NGD, F), whole(NINT, F),
            whole(NINT, F, F), whole(NINT, F),
            whole(NINT, FTW, F), whole(NINT, F),
            whole(NINT, F, F), whole(NINT, F),
            whole(NINT, F, F), whole(NINT, F, F), whole(NINT, F),
            whole(NINT, F, F), whole(NINT, F),
        ],
        out_specs=pl.BlockSpec((1, AT, F), lambda g: (g, 0, 0)),
        out_shape=jax.ShapeDtypeStruct((B, AT, F), f32),
    )(x0, positions, nbrd_b, nbrj_b, nbrk_b, nmask_b, tmask_b,
      jnp.asarray(_OFFCOL), jnp.asarray(_CTCOL), jnp.asarray(_STCOL),
      jnp.asarray(_EGT), jnp.asarray(_ETT),
      jnp.asarray(_REPT), jnp.asarray(_REPD),
      jnp.asarray(_SEGT).astype(bf16), jnp.asarray(_SEGD).astype(bf16),
      fd_W1, fd_b1, fd_W2, fd_b2, ft_W1, ft_b1, ft_W2, ft_b2,
      in2f_W, f2out_W, f2out_b, dense_W, dense_b)
    return out


# speedup vs baseline: 34.0653x; 1.0783x over previous
import functools

import numpy as np
import jax
import jax.numpy as jnp
from jax import lax
from jax.experimental import pallas as pl
from jax.experimental.pallas import tpu as pltpu
from jax.experimental.pallas import tpu_sc as plsc

B, AT, NBR, NBRT = 4, 128, 32, 96
F = 128
NGD = 25
NGT = 25
NTH = 10
ZETA = 8.0
CUTOFF = 6.0
NINT = 3
MAXZ = 100

BA = B * AT
RTB = AT * NBRT
RDB = AT * NBR
NCH = 2
RTC = RTB // NCH
FTW = NGT * NTH

_LOG2 = float(np.log(2.0))

_offs = np.linspace(0.001, CUTOFF - 0.5, NGT)
_W2 = float(_offs[1] - _offs[0]) ** 2
_OFFCOL = _offs.reshape(NGT, 1).astype(np.float32)
_theta = np.linspace(0.0, np.pi, NTH)
_CTCOL = np.cos(_theta).reshape(NTH, 1).astype(np.float32)
_STCOL = np.sin(_theta).reshape(NTH, 1).astype(np.float32)
_EGT = np.zeros((FTW, NGT), np.float32)
_ETT = np.zeros((FTW, NTH), np.float32)
for _g in range(NGT):
    for _t in range(NTH):
        _EGT[_g * NTH + _t, _g] = 1.0
        _ETT[_g * NTH + _t, _t] = 1.0
_REPT = np.kron(np.eye(AT, dtype=np.float32), np.ones((1, NBRT), np.float32))
_REPD = np.kron(np.eye(AT, dtype=np.float32), np.ones((1, NBR), np.float32))
_SEGT = _REPT.T.copy()
_SEGD = _REPD.T.copy()


def _ssp(x):
    return jnp.maximum(x, 0.0) + jnp.log1p(jnp.exp(-jnp.abs(x))) - _LOG2


def _onehot_t(idx_row, n, dtype=jnp.float32):
    io = jax.lax.broadcasted_iota(jnp.int32, (n, idx_row.shape[1]), 0)
    return (io == idx_row).astype(dtype)


def _cutoff(r):
    return 0.5 * (jnp.cos(r * (np.pi / CUTOFF)) + 1.0) * (r < CUTOFF).astype(r.dtype)


def _dot(a, b):
    return jnp.dot(a, b, preferred_element_type=jnp.float32)


_SC_NC = 2
_SC_NS = 16
_NW = _SC_NC * _SC_NS
_EPW = BA // _NW


def _embed_sc_body(atn_hbm, emb_hbm, out_hbm, idx_v, rows_v, sem):
    wid = lax.axis_index("s") * _SC_NC + lax.axis_index("c")
    base = wid * _EPW
    pltpu.sync_copy(atn_hbm.at[pl.ds(base, _EPW)], idx_v)
    pltpu.async_copy(emb_hbm.at[idx_v], rows_v, sem).wait()
    pltpu.sync_copy(rows_v, out_hbm.at[pl.ds(base, _EPW)])


def _embed_sc(atn_flat, emb):
    k = functools.partial(
        pl.kernel,
        mesh=plsc.VectorSubcoreMesh(core_axis_name="c", subcore_axis_name="s"),
        out_type=jax.ShapeDtypeStruct((BA, F), jnp.float32),
        scratch_types=[
            pltpu.VMEM((_EPW,), jnp.int32),
            pltpu.VMEM((_EPW, F), jnp.float32),
            pltpu.SemaphoreType.DMA,
        ],
    )(_embed_sc_body)
    return k(atn_flat, emb)


def _fused_body(x0_ref, pos_ref, nbrd_ref, nbrj_ref, nbrk_ref,
                nmask_ref, tmask_ref,
                offc_ref, ctc_ref, stc_ref, egt_ref, ett_ref,
                rept_ref, repd_ref, segt_ref, segd_ref,
                fdw1_ref, fdb1_ref, fdw2_ref, fdb2_ref,
                ftw1_ref, ftb1_ref, ftw2_ref, ftb2_ref,
                in2f_ref, f2ow_ref, f2ob_ref, dw_ref, db_ref,
                xo_ref):
    bf16 = jnp.bfloat16
    f32 = jnp.float32
    pos3 = pos_ref[0]
    pos_t = jnp.concatenate(
        [pos3, jnp.zeros((AT, 5), f32)], axis=1).T
    offc = offc_ref[...]

    bias_cols = jnp.concatenate(
        [fdb1_ref[...], fdb2_ref[...], ftb1_ref[...], ftb2_ref[...],
         f2ob_ref[...], db_ref[...]], axis=0).T

    def bcol(j):
        return bias_cols[:, j:j + 1]

    ohd = _onehot_t(nbrd_ref[0], AT)
    pj_d = _dot(pos_t, ohd)
    pi_d = _dot(pos_t, repd_ref[...])
    vd = pj_d - pi_d
    rdst = jnp.sqrt(jnp.sum(vd * vd, axis=0, keepdims=True) + 1e-9)
    sdd = rdst - offc
    fd_b = jnp.exp((-0.5 / _W2) * sdd * sdd).astype(bf16)
    cdf = _cutoff(rdst) * nmask_ref[0]

    ohj = _onehot_t(nbrj_ref[0], AT)
    ohk = _onehot_t(nbrk_ref[0], AT)
    pi_t = _dot(pos_t, rept_ref[...])
    vij = _dot(pos_t, ohj) - pi_t
    vik = _dot(pos_t, ohk) - pi_t
    rij = jnp.sqrt(jnp.sum(vij * vij, axis=0, keepdims=True) + 1e-9)
    rik = jnp.sqrt(jnp.sum(vik * vik, axis=0, keepdims=True) + 1e-9)
    cost = jnp.sum(vij * vik, axis=0, keepdims=True) / (rij * rik)
    cost = jnp.clip(cost, -1.0 + 1e-6, 1.0 - 1e-6)
    sint = jnp.sqrt(1.0 - cost * cost)
    sij = rij - offc
    sik = rik - offc
    gr_b = jnp.exp((-0.5 / _W2) * (sij * sij + sik * sik)).astype(bf16)
    base = 1.0 + ctc_ref[...] * cost + stc_ref[...] * sint
    b2 = base * base
    b4 = b2 * b2
    ang_b = ((b4 * b4) * (2.0 ** (1.0 - ZETA))).astype(bf16)
    rboth = jnp.concatenate([rij, rik], axis=0)
    cutb = _cutoff(rboth)
    ctf = cutb[0:1, :] * cutb[1:2, :] * tmask_ref[0]

    ohd_b = ohd.astype(bf16)
    ohj_b = ohj.astype(bf16)
    ohk_b = ohk.astype(bf16)
    egt_b = egt_ref[...].astype(bf16)
    ett_b = ett_ref[...].astype(bf16)
    segt_b = segt_ref[...]
    segd_b = segd_ref[...]

    x_t = x0_ref[...].T
    for i in range(NINT):
        fdw1t = fdw1_ref[i].T.astype(bf16)
        fdw2t = fdw2_ref[i].T.astype(bf16)
        ftw1t = ftw1_ref[i].T.astype(bf16)
        ftw2t = ftw2_ref[i].T.astype(bf16)
        in2ft = in2f_ref[i].T
        f2owt = f2ow_ref[i].T
        dwt = dw_ref[i].T
        y_t = _dot(in2ft, x_t)
        y_b = y_t.astype(bf16)
        hd = _ssp(_dot(fdw1t, fd_b) + bcol(i))
        wd = (_dot(fdw2t, hd.astype(bf16)) + bcol(NINT + i)) * cdf
        prod_d = (_dot(y_b, ohd_b) * wd).astype(bf16)
        agg = _dot(prod_d, segd_b)
        for c in range(NCH):
            lo, hi = c * RTC, (c + 1) * RTC
            feat = (_dot(egt_b, gr_b[:, lo:hi])
                    * _dot(ett_b, ang_b[:, lo:hi])).astype(bf16)
            ht = _ssp(_dot(ftw1t, feat) + bcol(2 * NINT + i))
            wt = (_dot(ftw2t, ht.astype(bf16))
                  + bcol(3 * NINT + i)) * ctf[:, lo:hi]
            prod_t = (_dot(y_b, ohj_b[:, lo:hi]) * _dot(y_b, ohk_b[:, lo:hi])
                      * wt).astype(bf16)
            agg = agg + _dot(prod_t, segt_b[lo:hi, :])
        v = _ssp(_dot(f2owt, agg) + bcol(4 * NINT + i))
        x_t = x_t + _dot(dwt, v) + bcol(5 * NINT + i)
    xo_ref[0] = x_t.T


def kernel(atomic_numbers, positions, neighbors, neighbor_mask, neighbors_j,
           neighbors_k, triple_mask, emb, fd_W1, fd_b1, fd_W2, fd_b2,
           ft_W1, ft_b1, ft_W2, ft_b2, in2f_W, f2out_W, f2out_b,
           dense_W, dense_b):
    f32 = jnp.float32
    bf16 = jnp.bfloat16
    nbrd_b = neighbors.astype(jnp.int32).reshape(B, 1, RDB)
    nbrj_b = neighbors_j.astype(jnp.int32).reshape(B, 1, RTB)
    nbrk_b = neighbors_k.astype(jnp.int32).reshape(B, 1, RTB)
    nmask_b = neighbor_mask.astype(f32).reshape(B, 1, RDB)
    tmask_b = triple_mask.astype(f32).reshape(B, 1, RTB)

    x0 = emb[atomic_numbers.astype(jnp.int32).reshape(BA)]

    whole = lambda *shape: pl.BlockSpec(shape, lambda g: tuple(0 for _ in shape))
    perb = lambda *shape: pl.BlockSpec((1,) + shape, lambda g: (g,) + tuple(
        0 for _ in shape))

    out = pl.pallas_call(
        _fused_body,
        grid=(B,),
        in_specs=[
            pl.BlockSpec((AT, F), lambda g: (g, 0)),
            perb(AT, 3),
            perb(1, RDB), perb(1, RTB), perb(1, RTB),
            perb(1, RDB), perb(1, RTB),
            whole(NGT, 1), whole(NTH, 1), whole(NTH, 1),
            whole(FTW, NGT), whole(FTW, NTH),
            whole(AT, RTB), whole(AT, RDB),
            whole(RTB, AT), whole(RDB, AT),
            whole(NINT, NGD, F), whole(NINT, F),
            whole(NINT, F, F), whole(NINT, F),
            whole(NINT, FTW, F), whole(NINT, F),
            whole(NINT, F, F), whole(NINT, F),
            whole(NINT, F, F), whole(NINT, F, F), whole(NINT, F),
            whole(NINT, F, F), whole(NINT, F),
        ],
        out_specs=pl.BlockSpec((1, AT, F), lambda g: (g, 0, 0)),
        out_shape=jax.ShapeDtypeStruct((B, AT, F), f32),
    )(x0, positions, nbrd_b, nbrj_b, nbrk_b, nmask_b, tmask_b,
      jnp.asarray(_OFFCOL), jnp.asarray(_CTCOL), jnp.asarray(_STCOL),
      jnp.asarray(_EGT), jnp.asarray(_ETT),
      jnp.asarray(_REPT), jnp.asarray(_REPD),
      jnp.asarray(_SEGT).astype(bf16), jnp.asarray(_SEGD).astype(bf16),
      fd_W1, fd_b1, fd_W2, fd_b2, ft_W1, ft_b1, ft_W2, ft_b2,
      in2f_W, f2out_W, f2out_b, dense_W, dense_b)
    return out


# speedup vs baseline: 39.8748x; 1.1705x over previous
import functools

import numpy as np
import jax
import jax.numpy as jnp
from jax import lax
from jax.experimental import pallas as pl
from jax.experimental.pallas import tpu as pltpu
from jax.experimental.pallas import tpu_sc as plsc

B, AT, NBR, NBRT = 4, 128, 32, 96
F = 128
NGD = 25
NGT = 25
NTH = 10
ZETA = 8.0
CUTOFF = 6.0
NINT = 3
MAXZ = 100

BA = B * AT
RTB = AT * NBRT
RDB = AT * NBR
NCH = 2
RTC = RTB // NCH
FTW = NGT * NTH

_LOG2 = float(np.log(2.0))

_offs = np.linspace(0.001, CUTOFF - 0.5, NGT)
_W2 = float(_offs[1] - _offs[0]) ** 2
_OFFCOL = _offs.reshape(NGT, 1).astype(np.float32)
_theta = np.linspace(0.0, np.pi, NTH)
_CTCOL = np.cos(_theta).reshape(NTH, 1).astype(np.float32)
_STCOL = np.sin(_theta).reshape(NTH, 1).astype(np.float32)
_EGT = np.zeros((FTW, NGT), np.float32)
_ETT = np.zeros((FTW, NTH), np.float32)
for _g in range(NGT):
    for _t in range(NTH):
        _EGT[_g * NTH + _t, _g] = 1.0
        _ETT[_g * NTH + _t, _t] = 1.0
_REPT = np.kron(np.eye(AT, dtype=np.float32), np.ones((1, NBRT), np.float32))
_REPD = np.kron(np.eye(AT, dtype=np.float32), np.ones((1, NBR), np.float32))
_SEGT = _REPT.T.copy()
_SEGD = _REPD.T.copy()


def _ssp(x):
    return jnp.maximum(x, 0.0) + jnp.log(1.0 + jnp.exp(-jnp.abs(x))) - _LOG2


def _cutoff(r):
    return 0.5 * (jnp.cos(r * (np.pi / CUTOFF)) + 1.0) * (r < CUTOFF).astype(r.dtype)


def _dot(a, b):
    return jnp.dot(a, b, preferred_element_type=jnp.float32)


_SC_NC = 2
_SC_NS = 16
_NW = _SC_NC * _SC_NS
_EPW = BA // _NW


def _embed_sc_body(atn_hbm, emb_hbm, out_hbm, idx_v, rows_v, sem):
    wid = lax.axis_index("s") * _SC_NC + lax.axis_index("c")
    base = wid * _EPW
    pltpu.sync_copy(atn_hbm.at[pl.ds(base, _EPW)], idx_v)
    pltpu.async_copy(emb_hbm.at[idx_v], rows_v, sem).wait()
    pltpu.sync_copy(rows_v, out_hbm.at[pl.ds(base, _EPW)])


def _embed_sc(atn_flat, emb):
    k = functools.partial(
        pl.kernel,
        mesh=plsc.VectorSubcoreMesh(core_axis_name="c", subcore_axis_name="s"),
        out_type=jax.ShapeDtypeStruct((BA, F), jnp.float32),
        scratch_types=[
            pltpu.VMEM((_EPW,), jnp.int32),
            pltpu.VMEM((_EPW, F), jnp.float32),
            pltpu.SemaphoreType.DMA,
        ],
    )(_embed_sc_body)
    return k(atn_flat, emb)


def _fused_body(x0_ref, pos_ref, nbrd_ref, nbrj_ref, nbrk_ref,
                nmask_ref, tmask_ref,
                offc_ref, ctc_ref, stc_ref, egt_ref, ett_ref,
                rept_ref, repd_ref, segt_ref, segd_ref,
                fdw1_ref, fdb1_ref, fdw2_ref, fdb2_ref,
                ftw1_ref, ftb1_ref, ftw2_ref, ftb2_ref,
                in2f_ref, f2ow_ref, f2ob_ref, dw_ref, db_ref,
                xo_ref):
    bf16 = jnp.bfloat16
    f32 = jnp.float32
    pos3 = pos_ref[0]
    pos_t = jnp.concatenate(
        [pos3, jnp.zeros((AT, 5), f32)], axis=1).T
    pos_hi = pos_t.astype(bf16)
    pos_lo = (pos_t - pos_hi.astype(f32)).astype(bf16)
    pos_hl = jnp.concatenate([pos_hi, pos_lo], axis=0)
    offc = offc_ref[...]

    io_t = jax.lax.broadcasted_iota(jnp.int32, (AT, RTB), 0)

    def onehot_b(idx_row, width):
        return (io_t[:, :width] == idx_row).astype(bf16)

    def gather_pos(oh_b):
        g = _dot(pos_hl, oh_b)
        return g[:8, :] + g[8:, :]

    bias_cols = jnp.concatenate(
        [fdb1_ref[...], fdb2_ref[...], ftb1_ref[...], ftb2_ref[...],
         f2ob_ref[...], db_ref[...]], axis=0).T

    def bcol(j):
        return bias_cols[:, j:j + 1]

    ohd_b = onehot_b(nbrd_ref[0], RDB)
    pj_d = gather_pos(ohd_b)
    pi_d = gather_pos(repd_ref[...])
    vd = pj_d - pi_d
    rdst = jnp.sqrt(jnp.sum(vd * vd, axis=0, keepdims=True) + 1e-9)
    sdd = rdst - offc
    fd_b = jnp.exp((-0.5 / _W2) * sdd * sdd).astype(bf16)
    cdf = _cutoff(rdst) * nmask_ref[0]

    ohj_b = onehot_b(nbrj_ref[0], RTB)
    ohk_b = onehot_b(nbrk_ref[0], RTB)
    pi_t = gather_pos(rept_ref[...])
    vij = gather_pos(ohj_b) - pi_t
    vik = gather_pos(ohk_b) - pi_t
    rij = jnp.sqrt(jnp.sum(vij * vij, axis=0, keepdims=True) + 1e-9)
    rik = jnp.sqrt(jnp.sum(vik * vik, axis=0, keepdims=True) + 1e-9)
    cost = jnp.sum(vij * vik, axis=0, keepdims=True) / (rij * rik)
    cost = jnp.clip(cost, -1.0 + 1e-6, 1.0 - 1e-6)
    sint = jnp.sqrt(1.0 - cost * cost)
    sij = rij - offc
    sik = rik - offc
    gr_b = jnp.exp((-0.5 / _W2) * (sij * sij + sik * sik)).astype(bf16)
    base = 1.0 + ctc_ref[...] * cost + stc_ref[...] * sint
    b2 = base * base
    b4 = b2 * b2
    ang_b = ((b4 * b4) * (2.0 ** (1.0 - ZETA))).astype(bf16)
    rboth = jnp.concatenate([rij, rik], axis=0)
    cutb = _cutoff(rboth)
    ctf = cutb[0:1, :] * cutb[1:2, :] * tmask_ref[0]

    egt_b = egt_ref[...].astype(bf16)
    ett_b = ett_ref[...].astype(bf16)
    segt_b = segt_ref[...]
    segd_b = segd_ref[...]

    x_t = x0_ref[...].T
    for i in range(NINT):
        fdw1t = fdw1_ref[i].T.astype(bf16)
        fdw2t = fdw2_ref[i].T.astype(bf16)
        ftw1t = ftw1_ref[i].T.astype(bf16)
        ftw2t = ftw2_ref[i].T.astype(bf16)
        in2ft = in2f_ref[i].T
        f2owt = f2ow_ref[i].T
        dwt = dw_ref[i].T
        y_t = _dot(in2ft, x_t)
        y_b = y_t.astype(bf16)
        hd = _ssp(_dot(fdw1t, fd_b) + bcol(i))
        wd = (_dot(fdw2t, hd.astype(bf16)) + bcol(NINT + i)) * cdf
        prod_d = (_dot(y_b, ohd_b) * wd).astype(bf16)
        agg = _dot(prod_d, segd_b)
        for c in range(NCH):
            lo, hi = c * RTC, (c + 1) * RTC
            feat = (_dot(egt_b, gr_b[:, lo:hi])
                    * _dot(ett_b, ang_b[:, lo:hi])).astype(bf16)
            ht = _ssp(_dot(ftw1t, feat) + bcol(2 * NINT + i))
            wt = (_dot(ftw2t, ht.astype(bf16))
                  + bcol(3 * NINT + i)) * ctf[:, lo:hi]
            prod_t = (_dot(y_b, ohj_b[:, lo:hi]) * _dot(y_b, ohk_b[:, lo:hi])
                      * wt).astype(bf16)
            agg = agg + _dot(prod_t, segt_b[lo:hi, :])
        v = _ssp(_dot(f2owt, agg) + bcol(4 * NINT + i))
        x_t = x_t + _dot(dwt, v) + bcol(5 * NINT + i)
    xo_ref[0] = x_t.T


def kernel(atomic_numbers, positions, neighbors, neighbor_mask, neighbors_j,
           neighbors_k, triple_mask, emb, fd_W1, fd_b1, fd_W2, fd_b2,
           ft_W1, ft_b1, ft_W2, ft_b2, in2f_W, f2out_W, f2out_b,
           dense_W, dense_b):
    f32 = jnp.float32
    bf16 = jnp.bfloat16
    nbrd_b = neighbors.astype(jnp.int32).reshape(B, 1, RDB)
    nbrj_b = neighbors_j.astype(jnp.int32).reshape(B, 1, RTB)
    nbrk_b = neighbors_k.astype(jnp.int32).reshape(B, 1, RTB)
    nmask_b = neighbor_mask.astype(f32).reshape(B, 1, RDB)
    tmask_b = triple_mask.astype(f32).reshape(B, 1, RTB)

    x0 = _embed_sc(atomic_numbers.astype(jnp.int32).reshape(BA),
                   emb.astype(f32))

    whole = lambda *shape: pl.BlockSpec(shape, lambda g: tuple(0 for _ in shape))
    perb = lambda *shape: pl.BlockSpec((1,) + shape, lambda g: (g,) + tuple(
        0 for _ in shape))

    out = pl.pallas_call(
        _fused_body,
        grid=(B,),
        in_specs=[
            pl.BlockSpec((AT, F), lambda g: (g, 0)),
            perb(AT, 3),
            perb(1, RDB), perb(1, RTB), perb(1, RTB),
            perb(1, RDB), perb(1, RTB),
            whole(NGT, 1), whole(NTH, 1), whole(NTH, 1),
            whole(FTW, NGT), whole(FTW, NTH),
            whole(AT, RTB), whole(AT, RDB),
            whole(RTB, AT), whole(RDB, AT),
            whole(NINT, NGD, F), whole(NINT, F),
            whole(NINT, F, F), whole(NINT, F),
            whole(NINT, FTW, F), whole(NINT, F),
            whole(NINT, F, F), whole(NINT, F),
            whole(NINT, F, F), whole(NINT, F, F), whole(NINT, F),
            whole(NINT, F, F), whole(NINT, F),
        ],
        out_specs=pl.BlockSpec((1, AT, F), lambda g: (g, 0, 0)),
        out_shape=jax.ShapeDtypeStruct((B, AT, F), f32),
    )(x0, positions, nbrd_b, nbrj_b, nbrk_b, nmask_b, tmask_b,
      jnp.asarray(_OFFCOL), jnp.asarray(_CTCOL), jnp.asarray(_STCOL),
      jnp.asarray(_EGT), jnp.asarray(_ETT),
      jnp.asarray(_REPT).astype(bf16), jnp.asarray(_REPD).astype(bf16),
      jnp.asarray(_SEGT).astype(bf16), jnp.asarray(_SEGD).astype(bf16),
      fd_W1, fd_b1, fd_W2, fd_b2, ft_W1, ft_b1, ft_W2, ft_b2,
      in2f_W, f2out_W, f2out_b, dense_W, dense_b)
    return out
